# baseline (speedup 1.0000x reference)
import functools

import jax
import jax.numpy as jnp
from jax import lax
from jax.experimental import pallas as pl
from jax.experimental.pallas import tpu as pltpu
from jax.experimental.pallas import tpu_sc as plsc

BATCH = 16384
EMBED_DIM = 64
LANES = 16

_info = plsc.get_sparse_core_info()
_NC = _info.num_cores
_NS = _info.num_subcores
NW = _NC * _NS
B_PER_W = BATCH // NW
HALF = B_PER_W // 2
NGROUP = HALF // LANES
VPR = EMBED_DIM // LANES


def _gmf_body(uidx_hbm, iidx_hbm, utab_hbm, itab_hbm, out_hbm,
              uidx_v, iidx_v, urows_v, irows_v, sem0, sem1, sem2, sem3):
    sems = (sem0, sem1, sem2, sem3)
    wid = lax.axis_index("s") * _NC + lax.axis_index("c")
    base = wid * B_PER_W

    pltpu.sync_copy(uidx_hbm.at[pl.ds(base, B_PER_W)], uidx_v)
    pltpu.sync_copy(iidx_hbm.at[pl.ds(base, B_PER_W)], iidx_v)

    def half_body(h, carry):
        def fire_group(g, c2):
            uvec = uidx_v[pl.ds(h * HALF + g * LANES, LANES)]
            ivec = iidx_v[pl.ds(h * HALF + g * LANES, LANES)]
            for l in range(LANES):
                pltpu.async_copy(
                    utab_hbm.at[pl.ds(uvec[l], 1)],
                    urows_v.at[pl.ds(g * LANES + l, 1)], sems[l % 2])
                pltpu.async_copy(
                    itab_hbm.at[pl.ds(ivec[l], 1)],
                    irows_v.at[pl.ds(g * LANES + l, 1)], sems[2 + l % 2])
            return c2

        lax.fori_loop(0, NGROUP, fire_group, None)

        quarter = HALF // 2
        pltpu.make_async_copy(
            utab_hbm.at[pl.ds(0, quarter)],
            urows_v.at[pl.ds(0, quarter)], sems[0]).wait()
        pltpu.make_async_copy(
            utab_hbm.at[pl.ds(0, quarter)],
            urows_v.at[pl.ds(0, quarter)], sems[1]).wait()
        pltpu.make_async_copy(
            itab_hbm.at[pl.ds(0, quarter)],
            irows_v.at[pl.ds(0, quarter)], sems[2]).wait()
        pltpu.make_async_copy(
            itab_hbm.at[pl.ds(0, quarter)],
            irows_v.at[pl.ds(0, quarter)], sems[3]).wait()

        def mul_row(i, c2):
            for c in range(VPR):
                sl = pl.ds(c * LANES, LANES)
                urows_v[i, sl] = urows_v[i, sl] * irows_v[i, sl]
            return c2

        lax.fori_loop(0, HALF, mul_row, None)

        pltpu.sync_copy(urows_v, out_hbm.at[pl.ds(base + h * HALF, HALF)])
        return carry

    lax.fori_loop(0, 2, half_body, None)


@jax.jit
def _gmf(uidx, iidx, utab, itab):
    mesh = plsc.VectorSubcoreMesh(core_axis_name="c", subcore_axis_name="s")
    kfn = functools.partial(
        pl.kernel,
        mesh=mesh,
        out_type=jax.ShapeDtypeStruct((BATCH, EMBED_DIM), jnp.float32),
        scratch_types=[
            pltpu.VMEM((B_PER_W,), jnp.int32),
            pltpu.VMEM((B_PER_W,), jnp.int32),
            pltpu.VMEM((HALF, EMBED_DIM), jnp.float32),
            pltpu.VMEM((HALF, EMBED_DIM), jnp.float32),
            pltpu.SemaphoreType.DMA,
            pltpu.SemaphoreType.DMA,
            pltpu.SemaphoreType.DMA,
            pltpu.SemaphoreType.DMA,
        ],
    )(_gmf_body)
    return kfn(uidx, iidx, utab, itab)


def kernel(user_indices, item_indices, user_table, item_table):
    uidx = user_indices.astype(jnp.int32)
    iidx = item_indices.astype(jnp.int32)
    return _gmf(uidx, iidx, user_table, item_table)


# speedup vs baseline: 1.0001x; 1.0001x over previous
import functools

import jax
import jax.numpy as jnp
from jax import lax
from jax.experimental import pallas as pl
from jax.experimental.pallas import tpu as pltpu
from jax.experimental.pallas import tpu_sc as plsc

BATCH = 16384
EMBED_DIM = 64
LANES = 16

_info = plsc.get_sparse_core_info()
_NC = _info.num_cores
_NS = _info.num_subcores
NW = _NC * _NS
B_PER_W = BATCH // NW
HALF = B_PER_W // 2
NGROUP = HALF // LANES
VPR = EMBED_DIM // LANES


def _gmf_body(uidx_hbm, iidx_hbm, utab_hbm, itab_hbm, out_hbm,
              uidx_v, iidx_v, urows_v, irows_v, sem):
    wid = lax.axis_index("s") * _NC + lax.axis_index("c")
    base = wid * B_PER_W

    pltpu.sync_copy(uidx_hbm.at[pl.ds(base, B_PER_W)], uidx_v)
    pltpu.sync_copy(iidx_hbm.at[pl.ds(base, B_PER_W)], iidx_v)

    def half_body(h, carry):
        def fire_group(g, c2):
            uvec = uidx_v[pl.ds(h * HALF + g * LANES, LANES)]
            ivec = iidx_v[pl.ds(h * HALF + g * LANES, LANES)]
            for l in range(LANES):
                pltpu.async_copy(
                    utab_hbm.at[pl.ds(uvec[l], 1)],
                    urows_v.at[pl.ds(g * LANES + l, 1)], sem)
                pltpu.async_copy(
                    itab_hbm.at[pl.ds(ivec[l], 1)],
                    irows_v.at[pl.ds(g * LANES + l, 1)], sem)
            return c2

        lax.fori_loop(0, NGROUP, fire_group, None)

        pltpu.make_async_copy(utab_hbm.at[pl.ds(0, HALF)], urows_v, sem).wait()
        pltpu.make_async_copy(itab_hbm.at[pl.ds(0, HALF)], irows_v, sem).wait()

        def mul_row(i, c2):
            for c in range(VPR):
                sl = pl.ds(c * LANES, LANES)
                urows_v[i, sl] = urows_v[i, sl] * irows_v[i, sl]
            return c2

        lax.fori_loop(0, HALF, mul_row, None)

        pltpu.sync_copy(urows_v, out_hbm.at[pl.ds(base + h * HALF, HALF)])
        return carry

    lax.fori_loop(0, 2, half_body, None)


@jax.jit
def _gmf(uidx, iidx, utab, itab):
    mesh = plsc.VectorSubcoreMesh(core_axis_name="c", subcore_axis_name="s")
    kfn = functools.partial(
        pl.kernel,
        mesh=mesh,
        out_type=jax.ShapeDtypeStruct((BATCH, EMBED_DIM), jnp.float32),
        scratch_types=[
            pltpu.VMEM((B_PER_W,), jnp.int32),
            pltpu.VMEM((B_PER_W,), jnp.int32),
            pltpu.VMEM((HALF, EMBED_DIM), jnp.float32),
            pltpu.VMEM((HALF, EMBED_DIM), jnp.float32),
            pltpu.SemaphoreType.DMA,
        ],
    )(_gmf_body)
    return kfn(uidx, iidx, utab, itab)


def kernel(user_indices, item_indices, user_table, item_table):
    uidx = user_indices.astype(jnp.int32)
    iidx = item_indices.astype(jnp.int32)
    return _gmf(uidx, iidx, user_table, item_table)


# speedup vs baseline: 1.0033x; 1.0032x over previous
import functools

import jax
import jax.numpy as jnp
from jax import lax
from jax.experimental import pallas as pl
from jax.experimental.pallas import tpu as pltpu
from jax.experimental.pallas import tpu_sc as plsc

BATCH = 16384
EMBED_DIM = 64
LANES = 16

_info = plsc.get_sparse_core_info()
_NC = _info.num_cores
_NS = _info.num_subcores
NW = _NC * _NS
B_PER_W = BATCH // NW
NSTAGE = 4
QUARTER = B_PER_W // NSTAGE
NGROUP = QUARTER // LANES
VPR = EMBED_DIM // LANES


def _gmf_body(uidx_hbm, iidx_hbm, utab_hbm, itab_hbm, out_hbm,
              uidx_v, iidx_v, ubuf0, ibuf0, ubuf1, ibuf1, obuf_v,
              sem0, sem1):
    wid = lax.axis_index("s") * _NC + lax.axis_index("c")
    base = wid * B_PER_W

    pltpu.sync_copy(uidx_hbm.at[pl.ds(base, B_PER_W)], uidx_v)
    pltpu.sync_copy(iidx_hbm.at[pl.ds(base, B_PER_W)], iidx_v)

    ubufs = (ubuf0, ubuf1)
    ibufs = (ibuf0, ibuf1)
    sems = (sem0, sem1)

    def fire_quarter(q, parity):
        ubuf = ubufs[parity]
        ibuf = ibufs[parity]
        sem = sems[parity]

        def fire_group(g, c2):
            uvec = uidx_v[pl.ds(q * QUARTER + g * LANES, LANES)]
            ivec = iidx_v[pl.ds(q * QUARTER + g * LANES, LANES)]
            for l in range(LANES):
                i = g * LANES + l
                pltpu.async_copy(
                    utab_hbm.at[pl.ds(uvec[l], 1)],
                    ubuf.at[pl.ds(i, 1)], sem)
                pltpu.async_copy(
                    itab_hbm.at[pl.ds(ivec[l], 1)],
                    ibuf.at[pl.ds(i, 1)], sem)
            return c2

        lax.fori_loop(0, NGROUP, fire_group, None)

    def drain_mul_write(q, parity):
        ubuf = ubufs[parity]
        ibuf = ibufs[parity]
        sem = sems[parity]
        pltpu.make_async_copy(utab_hbm.at[pl.ds(0, QUARTER)], ubuf, sem).wait()
        pltpu.make_async_copy(itab_hbm.at[pl.ds(0, QUARTER)], ibuf, sem).wait()

        def mul_row(i, c2):
            for c in range(VPR):
                sl = pl.ds(c * LANES, LANES)
                obuf_v[i, sl] = ubuf[i, sl] * ibuf[i, sl]
            return c2

        lax.fori_loop(0, QUARTER, mul_row, None)
        pltpu.sync_copy(obuf_v, out_hbm.at[pl.ds(base + q * QUARTER, QUARTER)])

    fire_quarter(0, 0)
    for q in range(1, NSTAGE):
        fire_quarter(q, q % 2)
        drain_mul_write(q - 1, (q - 1) % 2)
    drain_mul_write(NSTAGE - 1, (NSTAGE - 1) % 2)


@jax.jit
def _gmf(uidx, iidx, utab, itab):
    mesh = plsc.VectorSubcoreMesh(core_axis_name="c", subcore_axis_name="s")
    kfn = functools.partial(
        pl.kernel,
        mesh=mesh,
        out_type=jax.ShapeDtypeStruct((BATCH, EMBED_DIM), jnp.float32),
        scratch_types=[
            pltpu.VMEM((B_PER_W,), jnp.int32),
            pltpu.VMEM((B_PER_W,), jnp.int32),
            pltpu.VMEM((QUARTER, EMBED_DIM), jnp.float32),
            pltpu.VMEM((QUARTER, EMBED_DIM), jnp.float32),
            pltpu.VMEM((QUARTER, EMBED_DIM), jnp.float32),
            pltpu.VMEM((QUARTER, EMBED_DIM), jnp.float32),
            pltpu.VMEM((QUARTER, EMBED_DIM), jnp.float32),
            pltpu.SemaphoreType.DMA,
            pltpu.SemaphoreType.DMA,
        ],
    )(_gmf_body)
    return kfn(uidx, iidx, utab, itab)


def kernel(user_indices, item_indices, user_table, item_table):
    uidx = user_indices.astype(jnp.int32)
    iidx = item_indices.astype(jnp.int32)
    return _gmf(uidx, iidx, user_table, item_table)
